# baseline (device time: 22536 ns/iter reference)
import functools

import jax
import jax.numpy as jnp
from jax import lax
from jax.experimental import pallas as pl
from jax.experimental.pallas import tpu as pltpu

N_DEV = 4
N_TOK = 512
D_IN = 256
D_OUT = 512
N_EXP = 8
EXP_PER_DEV = 2
CHUNK = N_TOK // N_DEV


def kernel(x, router_W, route_idx, expert_W):
    def body(x_ref, rw_ref, idx_ref, ew_ref, out_ref,
             acc_ref, comm_ref, send_sems, recv_sems):
        my = lax.axis_index("i")
        left = lax.rem(my + N_DEV - 1, N_DEV)
        right = lax.rem(my + 1, N_DEV)

        xf = x_ref[:, :]
        scores = jnp.dot(xf, rw_ref[:, :], preferred_element_type=jnp.float32)
        s_max = jnp.max(scores, axis=-1, keepdims=True)
        p = jnp.exp(scores - s_max)
        probs = p / jnp.sum(p, axis=-1, keepdims=True)

        idx0 = idx_ref[:, 0:1]
        idx1 = idx_ref[:, 1:2]
        eids = lax.broadcasted_iota(jnp.int32, (N_TOK, N_EXP), 1)
        g0 = jnp.sum(jnp.where(eids == idx0, probs, 0.0), axis=1, keepdims=True)
        g1 = jnp.sum(jnp.where(eids == idx1, probs, 0.0), axis=1, keepdims=True)
        gs = g0 + g1

        xb = xf.astype(jnp.bfloat16)
        acc = jnp.zeros((N_TOK, D_OUT), jnp.float32)
        for j in range(EXP_PER_DEV):
            e = my * EXP_PER_DEV + j
            pe = jnp.sum(jnp.where(eids == e, probs, 0.0), axis=1, keepdims=True)
            hit = jnp.logical_or(idx0 == e, idx1 == e).astype(jnp.float32)
            w = pe / gs * hit
            y = jnp.dot(xb, ew_ref[j].astype(jnp.bfloat16),
                        preferred_element_type=jnp.float32)
            acc = acc + w * y
        acc_ref[:, :] = acc

        bar = pltpu.get_barrier_semaphore()
        for nbr in (left, right):
            pl.semaphore_signal(bar, inc=1, device_id=(nbr,),
                                device_id_type=pl.DeviceIdType.MESH)
        pl.semaphore_wait(bar, 2)

        for s in range(N_DEV - 1):
            send_chunk = lax.rem(my - (s + 1) + 2 * N_DEV, N_DEV)
            if s == 0:
                src = acc_ref.at[pl.ds(send_chunk * CHUNK, CHUNK), :]
            else:
                src = comm_ref.at[s - 1]
            rdma = pltpu.make_async_remote_copy(
                src_ref=src,
                dst_ref=comm_ref.at[s],
                send_sem=send_sems.at[s],
                recv_sem=recv_sems.at[s],
                device_id=(right,),
                device_id_type=pl.DeviceIdType.MESH,
            )
            rdma.start()
            rdma.wait()
            recv_chunk = lax.rem(my - (s + 2) + 2 * N_DEV, N_DEV)
            mine = acc_ref[pl.ds(recv_chunk * CHUNK, CHUNK), :]
            if s < N_DEV - 2:
                comm_ref[s] = comm_ref[s] + mine
            else:
                out_ref[:, :] = comm_ref[s] + mine

        @functools.partial(pl.run_scoped, sem=pltpu.SemaphoreType.REGULAR)
        def _(sem):
            for nbr in (left, right):
                pl.semaphore_signal(sem, inc=1, device_id=(nbr,),
                                    device_id_type=pl.DeviceIdType.MESH)
            pl.semaphore_wait(sem, 2)

    return pl.pallas_call(
        body,
        out_shape=jax.ShapeDtypeStruct((CHUNK, D_OUT), jnp.float32),
        in_specs=[pl.BlockSpec(memory_space=pltpu.VMEM)] * 4,
        out_specs=pl.BlockSpec(memory_space=pltpu.VMEM),
        scratch_shapes=[
            pltpu.VMEM((N_TOK, D_OUT), jnp.float32),
            pltpu.VMEM((N_DEV - 1, CHUNK, D_OUT), jnp.float32),
            pltpu.SemaphoreType.DMA((N_DEV - 1,)),
            pltpu.SemaphoreType.DMA((N_DEV - 1,)),
        ],
        compiler_params=pltpu.CompilerParams(collective_id=0),
    )(x, router_W, route_idx, expert_W)


# device time: 14614 ns/iter; 1.5421x vs baseline; 1.5421x over previous
import functools

import jax
import jax.numpy as jnp
from jax import lax
from jax.experimental import pallas as pl
from jax.experimental.pallas import tpu as pltpu

N_DEV = 4
N_TOK = 512
D_IN = 256
D_OUT = 512
N_EXP = 8
EXP_PER_DEV = 2
CHUNK = N_TOK // N_DEV


def kernel(x, router_W, route_idx, expert_W):
    def body(x_ref, rw_ref, idx_ref, ew_ref, out_ref,
             acc_ref, send_buf, recv_buf, send_sems, recv_sems):
        my = lax.axis_index("i")

        xf = x_ref[:, :]
        scores = jnp.dot(xf, rw_ref[:, :], preferred_element_type=jnp.float32)
        s_max = jnp.max(scores, axis=-1, keepdims=True)
        p = jnp.exp(scores - s_max)
        probs = p / jnp.sum(p, axis=-1, keepdims=True)

        idx0 = idx_ref[:, 0:1]
        idx1 = idx_ref[:, 1:2]
        eids = lax.broadcasted_iota(jnp.int32, (N_TOK, N_EXP), 1)
        g0 = jnp.sum(jnp.where(eids == idx0, probs, 0.0), axis=1, keepdims=True)
        g1 = jnp.sum(jnp.where(eids == idx1, probs, 0.0), axis=1, keepdims=True)
        gs = g0 + g1

        xb = xf.astype(jnp.bfloat16)
        acc = jnp.zeros((N_TOK, D_OUT), jnp.float32)
        for j in range(EXP_PER_DEV):
            e = my * EXP_PER_DEV + j
            pe = jnp.sum(jnp.where(eids == e, probs, 0.0), axis=1, keepdims=True)
            hit = jnp.logical_or(idx0 == e, idx1 == e).astype(jnp.float32)
            w = pe / gs * hit
            y = jnp.dot(xb, ew_ref[j].astype(jnp.bfloat16),
                        preferred_element_type=jnp.float32)
            acc = acc + w * y
        acc_ref[:, :] = acc

        for k in range(1, N_DEV):
            t = lax.rem(my + k, N_DEV)
            send_buf[k - 1] = acc_ref[pl.ds(t * CHUNK, CHUNK), :].astype(jnp.bfloat16)

        bar = pltpu.get_barrier_semaphore()
        for k in range(1, N_DEV):
            pl.semaphore_signal(bar, inc=1,
                                device_id=(lax.rem(my + k, N_DEV),),
                                device_id_type=pl.DeviceIdType.MESH)
        pl.semaphore_wait(bar, N_DEV - 1)

        rdmas = []
        for k in range(1, N_DEV):
            t = lax.rem(my + k, N_DEV)
            rdma = pltpu.make_async_remote_copy(
                src_ref=send_buf.at[k - 1],
                dst_ref=recv_buf.at[N_DEV - 1 - k],
                send_sem=send_sems.at[k - 1],
                recv_sem=recv_sems.at[N_DEV - 1 - k],
                device_id=(t,),
                device_id_type=pl.DeviceIdType.MESH,
            )
            rdma.start()
            rdmas.append(rdma)
        for rdma in rdmas:
            rdma.wait_recv()

        total = acc_ref[pl.ds(my * CHUNK, CHUNK), :]
        for j in range(N_DEV - 1):
            total = total + recv_buf[j].astype(jnp.float32)
        out_ref[:, :] = total

        for rdma in rdmas:
            rdma.wait_send()

        @functools.partial(pl.run_scoped, sem=pltpu.SemaphoreType.REGULAR)
        def _(sem):
            for k in range(1, N_DEV):
                pl.semaphore_signal(sem, inc=1,
                                    device_id=(lax.rem(my + k, N_DEV),),
                                    device_id_type=pl.DeviceIdType.MESH)
            pl.semaphore_wait(sem, N_DEV - 1)

    return pl.pallas_call(
        body,
        out_shape=jax.ShapeDtypeStruct((CHUNK, D_OUT), jnp.float32),
        in_specs=[pl.BlockSpec(memory_space=pltpu.VMEM)] * 4,
        out_specs=pl.BlockSpec(memory_space=pltpu.VMEM),
        scratch_shapes=[
            pltpu.VMEM((N_TOK, D_OUT), jnp.float32),
            pltpu.VMEM((N_DEV - 1, CHUNK, D_OUT), jnp.bfloat16),
            pltpu.VMEM((N_DEV - 1, CHUNK, D_OUT), jnp.bfloat16),
            pltpu.SemaphoreType.DMA((N_DEV - 1,)),
            pltpu.SemaphoreType.DMA((N_DEV - 1,)),
        ],
        compiler_params=pltpu.CompilerParams(collective_id=0),
    )(x, router_W, route_idx, expert_W)
